# Initial kernel scaffold; baseline (speedup 1.0000x reference)
import jax
import jax.numpy as jnp
from jax.experimental import pallas as pl


def kernel(x, edge_index_0, edge_index_1, Wq, bq, Wk, bk, Wv, bv, Ws, bs, bn_g, bn_b, bn_m, bn_v, Wo, bo):
    raise NotImplementedError("write your pallas kernel here")


# speedup vs baseline: 55.1163x; 55.1163x over previous
import functools

import jax
import jax.numpy as jnp
from jax import lax
from jax.experimental import pallas as pl
from jax.experimental.pallas import tpu as pltpu
from jax.experimental.pallas import tpu_sc as plsc

N0, N1, N2 = 100000, 20000, 2048
D0, D1 = 16, 16
DIM, OUTC, H = 128, 64, 2
K1SEL = D0 // 2

NC, NS = 2, 16
NW = NC * NS

@functools.lru_cache(maxsize=1)
def _sc_mesh():
    return plsc.VectorSubcoreMesh(core_axis_name="c", subcore_axis_name="s")


def _wid():
    return lax.axis_index("s") * NC + lax.axis_index("c")


def _logits_body(x_ref, w_ref, b_ref, logit_ref, sig_ref):
    z = x_ref[...] @ w_ref[...] + b_ref[...]
    logit_ref[...] = z
    sig_ref[...] = jax.nn.sigmoid(z)


def _logits0(x, Wo0, bo0):
    blk = 2000
    grid = (N0 // blk,)
    return pl.pallas_call(
        _logits_body,
        grid=grid,
        in_specs=[
            pl.BlockSpec((blk, DIM), lambda i: (i, 0)),
            pl.BlockSpec((DIM, 1), lambda i: (0, 0)),
            pl.BlockSpec((1, 1), lambda i: (0, 0)),
        ],
        out_specs=[
            pl.BlockSpec((blk, 1), lambda i: (i, 0)),
            pl.BlockSpec((blk, 1), lambda i: (i, 0)),
        ],
        out_shape=[
            jax.ShapeDtypeStruct((N0, 1), jnp.float32),
            jax.ShapeDtypeStruct((N0, 1), jnp.float32),
        ],
    )(x, Wo0, bo0.reshape(1, 1))


def _topk_body(l_hbm, src_hbm, out_hbm, l_v, src_v, sel_v, tmp_v):
    wid = _wid()
    gpw = N1 // NW
    pltpu.sync_copy(l_hbm, l_v)
    pltpu.sync_copy(src_hbm.at[pl.ds(wid * gpw * D0, gpw * D0)], src_v)

    eight = jnp.full((16,), K1SEL, jnp.int32)
    seven = jnp.full((16,), K1SEL - 1, jnp.int32)
    lanes = lax.iota(jnp.int32, 16)

    def body(g, _):
        idx = src_v[pl.ds(g * D0, 16)]
        row = plsc.load_gather(l_v, [idx])
        d = wid * gpw + g
        col = plsc.load_gather(l_v, [jnp.full((16,), d, jnp.int32)])
        sim = 1.0 - jnp.abs(row - col)
        skeys, _vals = plsc.sort_key_val(sim, lanes, descending=True)
        tmp_v[...] = skeys
        t = plsc.load_gather(tmp_v, [seven])
        gt = sim > t
        n_gt = plsc.all_reduce_population_count(gt)
        eq = sim == t
        csum = lax.cumsum(jnp.where(eq, 1, 0), axis=0)
        sel = jnp.logical_or(gt, jnp.logical_and(eq, csum <= (eight - n_gt)))
        plsc.store_compressed(sel_v.at[pl.ds(g * K1SEL, 16)], idx, mask=sel)
        return _

    lax.fori_loop(0, gpw, body, 0)
    opw = gpw * K1SEL
    pltpu.sync_copy(sel_v.at[pl.ds(0, opw)], out_hbm.at[pl.ds(wid * opw, opw)])


def _topk_select(l, src0):
    gpw = N1 // NW
    f = pl.kernel(
        _topk_body,
        mesh=_sc_mesh(),
        compiler_params=pltpu.CompilerParams(needs_layout_passes=False),
        out_type=jax.ShapeDtypeStruct((N1 * K1SEL,), jnp.int32),
        scratch_types=[
            pltpu.VMEM((N0,), jnp.float32),
            pltpu.VMEM((gpw * D0,), jnp.int32),
            pltpu.VMEM((gpw * K1SEL + 16,), jnp.int32),
            pltpu.VMEM((16,), jnp.float32),
        ],
    )
    return f(l, src0)


def _make_gather(n_rows_table, n_idx, chunk):
    bpw = n_idx // NW
    assert bpw % chunk == 0 and chunk % 8 == 0

    def body(table_hbm, idx_hbm, out_hbm, idx_v, rows_v, sem):
        wid = _wid()
        base = wid * bpw
        pltpu.sync_copy(idx_hbm.at[pl.ds(base, bpw)], idx_v)
        nch = bpw // chunk

        def step(c, _):
            pltpu.async_copy(
                table_hbm.at[idx_v.at[pl.ds(c * chunk, chunk)]], rows_v, sem
            ).wait()
            pltpu.sync_copy(rows_v, out_hbm.at[pl.ds(base + c * chunk, chunk), :])
            return _

        lax.fori_loop(0, nch, step, 0)

    def run(table, idx):
        f = pl.kernel(
            body,
            mesh=_sc_mesh(),
            compiler_params=pltpu.CompilerParams(needs_layout_passes=False),
            out_type=jax.ShapeDtypeStruct((n_idx, DIM), jnp.float32),
            scratch_types=[
                pltpu.VMEM((bpw,), jnp.int32),
                pltpu.VMEM((chunk, DIM), jnp.float32),
                pltpu.SemaphoreType.DMA,
            ],
        )
        return f(table, idx)

    return run


_gather_l1 = _make_gather(N0, N1 * K1SEL, 200)
_gather_l2 = _make_gather(N1, N2 * D1, 256)


def _attn_body(deg, xd_ref, rows_ref, wq_ref, wk_ref, wv_ref, ws_ref,
               bq_ref, bv_ref, bs_ref, bng_ref, bnb_ref, bnm_ref, bnv_ref,
               wo_ref, bo_ref, feat_ref, logit_ref):
    xd = xd_ref[...]
    bd = xd.shape[0]
    q = xd @ wq_ref[...] + bq_ref[...]
    rows = rows_ref[...]
    rows3 = rows.reshape(bd, deg, DIM)
    inv_sqrt = 1.0 / jnp.sqrt(float(OUTC))
    outs = []
    for h in range(H):
        sl = slice(h * OUTC, (h + 1) * OUTC)
        qh = q[:, sl]
        qWh = lax.dot_general(
            qh, wk_ref[...][:, sl], (((1,), (1,)), ((), ())))
        score = (rows3 * qWh[:, None, :]).sum(-1) * inv_sqrt
        m = score.max(axis=-1, keepdims=True)
        ex = jnp.exp(score - m)
        den = ex.sum(axis=-1, keepdims=True)
        alpha = ex / (den + 1e-16)
        agg = (alpha[:, :, None] * rows3).sum(1)
        outs.append(agg @ wv_ref[...][:, sl])
    out = jnp.concatenate(outs, axis=-1) + bv_ref[...]
    out = out + xd @ ws_ref[...] + bs_ref[...]
    z = (out - bnm_ref[...]) / jnp.sqrt(bnv_ref[...] + 1e-5)
    z = z * bng_ref[...] + bnb_ref[...]
    sp = jnp.maximum(z, 0.0) + jnp.log1p(jnp.exp(-jnp.abs(z)))
    feat = z * jnp.tanh(sp)
    feat_ref[...] = feat
    logit_ref[...] = feat @ wo_ref[...] + bo_ref[...]


def _attn_layer(n_dst, deg, blk, x_dst_src, rows, Wq, Wk, Wv, Ws, bq, bv, bs,
                bng, bnb, bnm, bnv, Wo, bo):
    grid = (n_dst // blk,)
    wspec = pl.BlockSpec((DIM, DIM), lambda i: (0, 0))
    bspec = pl.BlockSpec((1, DIM), lambda i: (0, 0))
    return pl.pallas_call(
        functools.partial(_attn_body, deg),
        grid=grid,
        in_specs=[
            pl.BlockSpec((blk, DIM), lambda i: (i, 0)),
            pl.BlockSpec((blk * deg, DIM), lambda i: (i, 0)),
            wspec, wspec, wspec, wspec,
            bspec, bspec, bspec, bspec, bspec, bspec, bspec,
            pl.BlockSpec((DIM, 1), lambda i: (0, 0)),
            pl.BlockSpec((1, 1), lambda i: (0, 0)),
        ],
        out_specs=[
            pl.BlockSpec((blk, DIM), lambda i: (i, 0)),
            pl.BlockSpec((blk, 1), lambda i: (i, 0)),
        ],
        out_shape=[
            jax.ShapeDtypeStruct((n_dst, DIM), jnp.float32),
            jax.ShapeDtypeStruct((n_dst, 1), jnp.float32),
        ],
    )(x_dst_src, rows, Wq, Wk, Wv, Ws,
      bq.reshape(1, DIM), bv.reshape(1, DIM), bs.reshape(1, DIM),
      bng.reshape(1, DIM), bnb.reshape(1, DIM), bnm.reshape(1, DIM),
      bnv.reshape(1, DIM), Wo, bo.reshape(1, 1))


def kernel(x, edge_index_0, edge_index_1, Wq, bq, Wk, bk, Wv, bv, Ws, bs,
           bn_g, bn_b, bn_m, bn_v, Wo, bo):
    src0 = edge_index_0[0]
    src1 = edge_index_1[0]

    logits0, l = _logits0(x, Wo[0], bo[0])
    t0 = logits0[:N2]

    sel_src = _topk_select(l.reshape(-1), src0)
    rows1 = _gather_l1(x, sel_src)

    feat1, logits1 = _attn_layer(
        N1, K1SEL, 400, x[:N1], rows1, Wq[0], Wk[0], Wv[0], Ws[0],
        bq[0], bv[0], bs[0], bn_g[0], bn_b[0], bn_m[0], bn_v[0], Wo[1], bo[1])
    t1 = logits1[:N2]

    rows2 = _gather_l2(feat1, src1)
    _feat2, logits2 = _attn_layer(
        N2, D1, 256, feat1[:N2], rows2, Wq[1], Wk[1], Wv[1], Ws[1],
        bq[1], bv[1], bs[1], bn_g[1], bn_b[1], bn_m[1], bn_v[1], Wo[2], bo[2])
    return (t0, t1, logits2)


# speedup vs baseline: 65.4033x; 1.1866x over previous
import functools

import jax
import jax.numpy as jnp
from jax import lax
from jax.experimental import pallas as pl
from jax.experimental.pallas import tpu as pltpu
from jax.experimental.pallas import tpu_sc as plsc

N0, N1, N2 = 100000, 20000, 2048
D0, D1 = 16, 16
DIM, OUTC, H = 128, 64, 2
K1SEL = D0 // 2

NC, NS = 2, 16
NW = NC * NS

@functools.lru_cache(maxsize=1)
def _sc_mesh():
    return plsc.VectorSubcoreMesh(core_axis_name="c", subcore_axis_name="s")


def _wid():
    return lax.axis_index("s") * NC + lax.axis_index("c")


def _logits_body(x_ref, w_ref, b_ref, logit_ref, sig_ref):
    z = x_ref[...] @ w_ref[...] + b_ref[...]
    logit_ref[...] = z
    sig_ref[...] = jax.nn.sigmoid(z)


def _logits0(x, Wo0, bo0):
    blk = 2000
    grid = (N0 // blk,)
    return pl.pallas_call(
        _logits_body,
        grid=grid,
        in_specs=[
            pl.BlockSpec((blk, DIM), lambda i: (i, 0)),
            pl.BlockSpec((DIM, 1), lambda i: (0, 0)),
            pl.BlockSpec((1, 1), lambda i: (0, 0)),
        ],
        out_specs=[
            pl.BlockSpec((blk, 1), lambda i: (i, 0)),
            pl.BlockSpec((blk, 1), lambda i: (i, 0)),
        ],
        out_shape=[
            jax.ShapeDtypeStruct((N0, 1), jnp.float32),
            jax.ShapeDtypeStruct((N0, 1), jnp.float32),
        ],
    )(x, Wo0, bo0.reshape(1, 1))


def _topk_body(l_hbm, src_hbm, out_hbm, l_v, src_v, sel_v, tmp_v):
    wid = _wid()
    gpw = N1 // NW
    pltpu.sync_copy(l_hbm, l_v)
    pltpu.sync_copy(src_hbm.at[pl.ds(wid * gpw * D0, gpw * D0)], src_v)

    eight = jnp.full((16,), K1SEL, jnp.int32)
    seven = jnp.full((16,), K1SEL - 1, jnp.int32)
    lanes = lax.iota(jnp.int32, 16)

    def body(g, _):
        idx = src_v[pl.ds(g * D0, 16)]
        row = plsc.load_gather(l_v, [idx])
        d = wid * gpw + g
        col = plsc.load_gather(l_v, [jnp.full((16,), d, jnp.int32)])
        sim = 1.0 - jnp.abs(row - col)
        skeys, _vals = plsc.sort_key_val(sim, lanes, descending=True)
        tmp_v[...] = skeys
        t = plsc.load_gather(tmp_v, [seven])
        gt = sim > t
        n_gt = plsc.all_reduce_population_count(gt)
        eq = sim == t
        csum = lax.cumsum(jnp.where(eq, 1, 0), axis=0)
        sel = jnp.logical_or(gt, jnp.logical_and(eq, csum <= (eight - n_gt)))
        plsc.store_compressed(sel_v.at[pl.ds(g * K1SEL, 16)], idx, mask=sel)
        return _

    lax.fori_loop(0, gpw, body, 0)
    opw = gpw * K1SEL
    pltpu.sync_copy(sel_v.at[pl.ds(0, opw)], out_hbm.at[pl.ds(wid * opw, opw)])


def _topk_select(l, src0):
    gpw = N1 // NW
    f = pl.kernel(
        _topk_body,
        mesh=_sc_mesh(),
        compiler_params=pltpu.CompilerParams(needs_layout_passes=False),
        out_type=jax.ShapeDtypeStruct((N1 * K1SEL,), jnp.int32),
        scratch_types=[
            pltpu.VMEM((N0,), jnp.float32),
            pltpu.VMEM((gpw * D0,), jnp.int32),
            pltpu.VMEM((gpw * K1SEL + 16,), jnp.int32),
            pltpu.VMEM((16,), jnp.float32),
        ],
    )
    return f(l, src0)


def _make_gather(n_rows_table, n_idx, chunk):
    bpw = n_idx // NW
    assert bpw % chunk == 0 and chunk % 8 == 0

    nch = bpw // chunk

    def body(table_hbm, idx_hbm, out_hbm, idx_v, rows_a, rows_b, sem_a, sem_b):
        wid = _wid()
        base = wid * bpw
        pltpu.sync_copy(idx_hbm.at[pl.ds(base, bpw)], idx_v)

        def gather(c, buf, sem):
            pltpu.async_copy(
                table_hbm.at[idx_v.at[pl.ds(c * chunk, chunk)]], buf, sem)

        def gwait(c, buf, sem):
            pltpu.make_async_copy(
                table_hbm.at[idx_v.at[pl.ds(c * chunk, chunk)]], buf, sem
            ).wait()

        gather(0, rows_a, sem_a)

        def step(c, carry):
            even = (c % 2) == 0

            @pl.when(jnp.logical_and(c + 1 < nch, even))
            def _pf_b():
                gather(c + 1, rows_b, sem_b)

            @pl.when(jnp.logical_and(c + 1 < nch, jnp.logical_not(even)))
            def _pf_a():
                gather(c + 1, rows_a, sem_a)

            @pl.when(even)
            def _drain_a():
                gwait(c, rows_a, sem_a)
                pltpu.sync_copy(
                    rows_a, out_hbm.at[pl.ds(base + c * chunk, chunk), :])

            @pl.when(jnp.logical_not(even))
            def _drain_b():
                gwait(c, rows_b, sem_b)
                pltpu.sync_copy(
                    rows_b, out_hbm.at[pl.ds(base + c * chunk, chunk), :])
            return carry

        lax.fori_loop(0, nch, step, 0)

    def run(table, idx):
        f = pl.kernel(
            body,
            mesh=_sc_mesh(),
            compiler_params=pltpu.CompilerParams(needs_layout_passes=False),
            out_type=jax.ShapeDtypeStruct((n_idx, DIM), jnp.float32),
            scratch_types=[
                pltpu.VMEM((bpw,), jnp.int32),
                pltpu.VMEM((chunk, DIM), jnp.float32),
                pltpu.VMEM((chunk, DIM), jnp.float32),
                pltpu.SemaphoreType.DMA,
                pltpu.SemaphoreType.DMA,
            ],
        )
        return f(table, idx)

    return run


_gather_l1 = _make_gather(N0, N1 * K1SEL, 200)
_gather_l2 = _make_gather(N1, N2 * D1, 256)


def _attn_body(deg, xd_ref, rows_ref, wq_ref, wk_ref, wv_ref, ws_ref,
               bq_ref, bv_ref, bs_ref, bng_ref, bnb_ref, bnm_ref, bnv_ref,
               wo_ref, bo_ref, feat_ref, logit_ref):
    xd = xd_ref[...]
    bd = xd.shape[0]
    q = xd @ wq_ref[...] + bq_ref[...]
    rows = rows_ref[...]
    rows3 = rows.reshape(bd, deg, DIM)
    vv = rows @ wv_ref[...]
    inv_sqrt = 1.0 / jnp.sqrt(float(OUTC))
    scores = []
    for h in range(H):
        sl = slice(h * OUTC, (h + 1) * OUTC)
        qh = q[:, sl]
        qWh = lax.dot_general(
            qh, wk_ref[...][:, sl], (((1,), (1,)), ((), ())))
        scores.append((rows3 * qWh[:, None, :]).sum(-1))
    s01 = jnp.concatenate(scores, axis=-1) * inv_sqrt
    m0 = s01[:, :deg].max(axis=-1, keepdims=True)
    m1 = s01[:, deg:].max(axis=-1, keepdims=True)
    mC = jnp.concatenate([jnp.broadcast_to(m0, (bd, deg)),
                          jnp.broadcast_to(m1, (bd, deg))], axis=-1)
    ex = jnp.exp(s01 - mC)
    d0 = ex[:, :deg].sum(axis=-1, keepdims=True)
    d1 = ex[:, deg:].sum(axis=-1, keepdims=True)
    dC = jnp.concatenate([jnp.broadcast_to(d0, (bd, deg)),
                          jnp.broadcast_to(d1, (bd, deg))], axis=-1)
    alpha = ex / (dC + 1e-16)
    outs = []
    for h in range(H):
        sl = slice(h * OUTC, (h + 1) * OUTC)
        ah = alpha[:, h * deg:(h + 1) * deg]
        vvh = vv[:, sl].reshape(bd, deg, OUTC)
        outs.append((ah[:, :, None] * vvh).sum(1))
    out = jnp.concatenate(outs, axis=-1) + bv_ref[...]
    out = out + xd @ ws_ref[...] + bs_ref[...]
    z = (out - bnm_ref[...]) / jnp.sqrt(bnv_ref[...] + 1e-5)
    z = z * bng_ref[...] + bnb_ref[...]
    sp = jnp.maximum(z, 0.0) + jnp.log1p(jnp.exp(-jnp.abs(z)))
    feat = z * jnp.tanh(sp)
    feat_ref[...] = feat
    logit_ref[...] = feat @ wo_ref[...] + bo_ref[...]


def _attn_layer(n_dst, deg, blk, x_dst_src, rows, Wq, Wk, Wv, Ws, bq, bv, bs,
                bng, bnb, bnm, bnv, Wo, bo):
    grid = (n_dst // blk,)
    wspec = pl.BlockSpec((DIM, DIM), lambda i: (0, 0))
    bspec = pl.BlockSpec((1, DIM), lambda i: (0, 0))
    return pl.pallas_call(
        functools.partial(_attn_body, deg),
        grid=grid,
        in_specs=[
            pl.BlockSpec((blk, DIM), lambda i: (i, 0)),
            pl.BlockSpec((blk * deg, DIM), lambda i: (i, 0)),
            wspec, wspec, wspec, wspec,
            bspec, bspec, bspec, bspec, bspec, bspec, bspec,
            pl.BlockSpec((DIM, 1), lambda i: (0, 0)),
            pl.BlockSpec((1, 1), lambda i: (0, 0)),
        ],
        out_specs=[
            pl.BlockSpec((blk, DIM), lambda i: (i, 0)),
            pl.BlockSpec((blk, 1), lambda i: (i, 0)),
        ],
        out_shape=[
            jax.ShapeDtypeStruct((n_dst, DIM), jnp.float32),
            jax.ShapeDtypeStruct((n_dst, 1), jnp.float32),
        ],
    )(x_dst_src, rows, Wq, Wk, Wv, Ws,
      bq.reshape(1, DIM), bv.reshape(1, DIM), bs.reshape(1, DIM),
      bng.reshape(1, DIM), bnb.reshape(1, DIM), bnm.reshape(1, DIM),
      bnv.reshape(1, DIM), Wo, bo.reshape(1, 1))


def kernel(x, edge_index_0, edge_index_1, Wq, bq, Wk, bk, Wv, bv, Ws, bs,
           bn_g, bn_b, bn_m, bn_v, Wo, bo):
    src0 = edge_index_0[0]
    src1 = edge_index_1[0]

    logits0, l = _logits0(x, Wo[0], bo[0])
    t0 = logits0[:N2]

    sel_src = _topk_select(l.reshape(-1), src0)
    rows1 = _gather_l1(x, sel_src)

    feat1, logits1 = _attn_layer(
        N1, K1SEL, 400, x[:N1], rows1, Wq[0], Wk[0], Wv[0], Ws[0],
        bq[0], bv[0], bs[0], bn_g[0], bn_b[0], bn_m[0], bn_v[0], Wo[1], bo[1])
    t1 = logits1[:N2]

    rows2 = _gather_l2(feat1, src1)
    _feat2, logits2 = _attn_layer(
        N2, D1, 256, feat1[:N2], rows2, Wq[1], Wk[1], Wv[1], Ws[1],
        bq[1], bv[1], bs[1], bn_g[1], bn_b[1], bn_m[1], bn_v[1], Wo[2], bo[2])
    return (t0, t1, logits2)


# speedup vs baseline: 74.2656x; 1.1355x over previous
import functools

import jax
import jax.numpy as jnp
from jax import lax
from jax.experimental import pallas as pl
from jax.experimental.pallas import tpu as pltpu
from jax.experimental.pallas import tpu_sc as plsc

N0, N1, N2 = 100000, 20000, 2048
N0P = 100352
D0, D1 = 16, 16
DIM, OUTC, H = 128, 64, 2
K1SEL = D0 // 2

NC, NS = 2, 16
NW = NC * NS

@functools.lru_cache(maxsize=1)
def _sc_mesh():
    return plsc.VectorSubcoreMesh(core_axis_name="c", subcore_axis_name="s")


def _wid():
    return lax.axis_index("s") * NC + lax.axis_index("c")


def _logits_body(blk, x_ref, wt_ref, b_ref, logit_ref, sig_ref):
    i = pl.program_id(0)
    z = lax.dot_general(wt_ref[...], x_ref[...],
                        (((1,), (1,)), ((), ()))) + b_ref[...]
    logit_ref[:, pl.ds(i * blk, blk)] = z
    sig_ref[:, pl.ds(i * blk, blk)] = jax.nn.sigmoid(z)


def _logits0(x, Wo0, bo0):
    blk = 2048
    grid = (N0P // blk,)
    return pl.pallas_call(
        functools.partial(_logits_body, blk),
        grid=grid,
        in_specs=[
            pl.BlockSpec((blk, DIM), lambda i: (i, 0)),
            pl.BlockSpec((1, DIM), lambda i: (0, 0)),
            pl.BlockSpec((1, 1), lambda i: (0, 0)),
        ],
        out_specs=[
            pl.BlockSpec((1, N0P), lambda i: (0, 0)),
            pl.BlockSpec((1, N0P), lambda i: (0, 0)),
        ],
        out_shape=[
            jax.ShapeDtypeStruct((1, N0P), jnp.float32),
            jax.ShapeDtypeStruct((1, N0P), jnp.float32),
        ],
    )(x, Wo0.reshape(1, DIM), bo0.reshape(1, 1))


def _topk_body(l_hbm, src_hbm, out_hbm, l_v, src_v, sel_v, tmp_v):
    wid = _wid()
    gpw = N1 // NW
    pltpu.sync_copy(l_hbm, l_v)
    pltpu.sync_copy(src_hbm.at[pl.ds(wid * gpw * D0, gpw * D0)], src_v)

    eight = jnp.full((16,), K1SEL, jnp.int32)
    seven = jnp.full((16,), K1SEL - 1, jnp.int32)
    lanes = lax.iota(jnp.int32, 16)

    def body(g, _):
        idx = src_v[pl.ds(g * D0, 16)]
        row = plsc.load_gather(l_v, [idx])
        d = wid * gpw + g
        col = plsc.load_gather(l_v, [jnp.full((16,), d, jnp.int32)])
        sim = 1.0 - jnp.abs(row - col)
        skeys, _vals = plsc.sort_key_val(sim, lanes, descending=True)
        tmp_v[...] = skeys
        t = plsc.load_gather(tmp_v, [seven])
        gt = sim > t
        n_gt = plsc.all_reduce_population_count(gt)
        eq = sim == t
        csum = lax.cumsum(jnp.where(eq, 1, 0), axis=0)
        sel = jnp.logical_or(gt, jnp.logical_and(eq, csum <= (eight - n_gt)))
        plsc.store_compressed(sel_v.at[pl.ds(g * K1SEL, 16)], idx, mask=sel)
        return _

    lax.fori_loop(0, gpw, body, 0)
    opw = gpw * K1SEL
    pltpu.sync_copy(sel_v.at[pl.ds(0, opw)], out_hbm.at[pl.ds(wid * opw, opw)])


def _topk_select(l, src0):
    gpw = N1 // NW
    f = pl.kernel(
        _topk_body,
        mesh=_sc_mesh(),
        compiler_params=pltpu.CompilerParams(needs_layout_passes=False),
        out_type=jax.ShapeDtypeStruct((N1 * K1SEL,), jnp.int32),
        scratch_types=[
            pltpu.VMEM((N0P,), jnp.float32),
            pltpu.VMEM((gpw * D0,), jnp.int32),
            pltpu.VMEM((gpw * K1SEL + 16,), jnp.int32),
            pltpu.VMEM((16,), jnp.float32),
        ],
    )
    return f(l, src0)


def _make_gather(n_rows_table, n_idx, chunk):
    bpw = n_idx // NW
    assert bpw % chunk == 0 and chunk % 8 == 0

    nch = bpw // chunk

    def body(table_hbm, idx_hbm, out_hbm, idx_v, rows_a, rows_b, sem_a, sem_b):
        wid = _wid()
        base = wid * bpw
        pltpu.sync_copy(idx_hbm.at[pl.ds(base, bpw)], idx_v)

        def gather(c, buf, sem):
            pltpu.async_copy(
                table_hbm.at[idx_v.at[pl.ds(c * chunk, chunk)]], buf, sem)

        def gwait(c, buf, sem):
            pltpu.make_async_copy(
                table_hbm.at[idx_v.at[pl.ds(c * chunk, chunk)]], buf, sem
            ).wait()

        gather(0, rows_a, sem_a)

        def step(c, carry):
            even = (c % 2) == 0

            @pl.when(jnp.logical_and(c + 1 < nch, even))
            def _pf_b():
                gather(c + 1, rows_b, sem_b)

            @pl.when(jnp.logical_and(c + 1 < nch, jnp.logical_not(even)))
            def _pf_a():
                gather(c + 1, rows_a, sem_a)

            @pl.when(even)
            def _drain_a():
                gwait(c, rows_a, sem_a)
                pltpu.sync_copy(
                    rows_a, out_hbm.at[pl.ds(base + c * chunk, chunk), :])

            @pl.when(jnp.logical_not(even))
            def _drain_b():
                gwait(c, rows_b, sem_b)
                pltpu.sync_copy(
                    rows_b, out_hbm.at[pl.ds(base + c * chunk, chunk), :])
            return carry

        lax.fori_loop(0, nch, step, 0)

    def run(table, idx):
        f = pl.kernel(
            body,
            mesh=_sc_mesh(),
            compiler_params=pltpu.CompilerParams(needs_layout_passes=False),
            out_type=jax.ShapeDtypeStruct((n_idx, DIM), jnp.float32),
            scratch_types=[
                pltpu.VMEM((bpw,), jnp.int32),
                pltpu.VMEM((chunk, DIM), jnp.float32),
                pltpu.VMEM((chunk, DIM), jnp.float32),
                pltpu.SemaphoreType.DMA,
                pltpu.SemaphoreType.DMA,
            ],
        )
        return f(table, idx)

    return run


_gather_l1 = _make_gather(N0, N1 * K1SEL, 200)
_gather_l2 = _make_gather(N1, N2 * D1, 256)


def _attn_body(deg, xd_ref, rows_ref, wq_ref, wk_ref, wv_ref, ws_ref,
               bq_ref, bv_ref, bs_ref, bng_ref, bnb_ref, bnm_ref, bnv_ref,
               wo_ref, bo_ref, feat_ref, logit_ref):
    xd = xd_ref[...]
    bd = xd.shape[0]
    q = xd @ wq_ref[...] + bq_ref[...]
    rows = rows_ref[...]
    rows3 = rows.reshape(bd, deg, DIM)
    vv = rows @ wv_ref[...]
    inv_sqrt = 1.0 / jnp.sqrt(float(OUTC))
    scores = []
    for h in range(H):
        sl = slice(h * OUTC, (h + 1) * OUTC)
        qh = q[:, sl]
        qWh = lax.dot_general(
            qh, wk_ref[...][:, sl], (((1,), (1,)), ((), ())))
        scores.append((rows3 * qWh[:, None, :]).sum(-1))
    s01 = jnp.concatenate(scores, axis=-1) * inv_sqrt
    m0 = s01[:, :deg].max(axis=-1, keepdims=True)
    m1 = s01[:, deg:].max(axis=-1, keepdims=True)
    mC = jnp.concatenate([jnp.broadcast_to(m0, (bd, deg)),
                          jnp.broadcast_to(m1, (bd, deg))], axis=-1)
    ex = jnp.exp(s01 - mC)
    d0 = ex[:, :deg].sum(axis=-1, keepdims=True)
    d1 = ex[:, deg:].sum(axis=-1, keepdims=True)
    dC = jnp.concatenate([jnp.broadcast_to(d0, (bd, deg)),
                          jnp.broadcast_to(d1, (bd, deg))], axis=-1)
    alpha = ex / (dC + 1e-16)
    outs = []
    for h in range(H):
        sl = slice(h * OUTC, (h + 1) * OUTC)
        ah = alpha[:, h * deg:(h + 1) * deg]
        vvh = vv[:, sl].reshape(bd, deg, OUTC)
        outs.append((ah[:, :, None] * vvh).sum(1))
    out = jnp.concatenate(outs, axis=-1) + bv_ref[...]
    out = out + xd @ ws_ref[...] + bs_ref[...]
    z = (out - bnm_ref[...]) / jnp.sqrt(bnv_ref[...] + 1e-5)
    z = z * bng_ref[...] + bnb_ref[...]
    sp = jnp.maximum(z, 0.0) + jnp.log1p(jnp.exp(-jnp.abs(z)))
    feat = z * jnp.tanh(sp)
    feat_ref[...] = feat
    logit_ref[:, pl.ds(pl.program_id(0) * bd, bd)] = lax.dot_general(
        wo_ref[...], feat, (((1,), (1,)), ((), ()))) + bo_ref[...]


def _attn_layer(n_dst, deg, blk, x_dst_src, rows, Wq, Wk, Wv, Ws, bq, bv, bs,
                bng, bnb, bnm, bnv, Wo, bo):
    nblk = (n_dst + blk - 1) // blk
    npad = nblk * blk
    grid = (nblk,)
    wspec = pl.BlockSpec((DIM, DIM), lambda i: (0, 0))
    bspec = pl.BlockSpec((1, DIM), lambda i: (0, 0))
    return pl.pallas_call(
        functools.partial(_attn_body, deg),
        grid=grid,
        in_specs=[
            pl.BlockSpec((blk, DIM), lambda i: (i, 0)),
            pl.BlockSpec((blk * deg, DIM), lambda i: (i, 0)),
            wspec, wspec, wspec, wspec,
            bspec, bspec, bspec, bspec, bspec, bspec, bspec,
            pl.BlockSpec((1, DIM), lambda i: (0, 0)),
            pl.BlockSpec((1, 1), lambda i: (0, 0)),
        ],
        out_specs=[
            pl.BlockSpec((blk, DIM), lambda i: (i, 0)),
            pl.BlockSpec((1, npad), lambda i: (0, 0)),
        ],
        out_shape=[
            jax.ShapeDtypeStruct((n_dst, DIM), jnp.float32),
            jax.ShapeDtypeStruct((1, npad), jnp.float32),
        ],
    )(x_dst_src, rows, Wq, Wk, Wv, Ws,
      bq.reshape(1, DIM), bv.reshape(1, DIM), bs.reshape(1, DIM),
      bng.reshape(1, DIM), bnb.reshape(1, DIM), bnm.reshape(1, DIM),
      bnv.reshape(1, DIM), Wo.reshape(1, DIM), bo.reshape(1, 1))


def kernel(x, edge_index_0, edge_index_1, Wq, bq, Wk, bk, Wv, bv, Ws, bs,
           bn_g, bn_b, bn_m, bn_v, Wo, bo):
    src0 = edge_index_0[0]
    src1 = edge_index_1[0]

    logits0, l = _logits0(x, Wo[0], bo[0])
    t0 = logits0[0, :N2].reshape(N2, 1)

    sel_src = _topk_select(l.reshape(-1), src0)
    rows1 = _gather_l1(x, sel_src)

    feat1, logits1 = _attn_layer(
        N1, K1SEL, 512, x[:N1], rows1, Wq[0], Wk[0], Wv[0], Ws[0],
        bq[0], bv[0], bs[0], bn_g[0], bn_b[0], bn_m[0], bn_v[0], Wo[1], bo[1])
    t1 = logits1[0, :N2].reshape(N2, 1)

    rows2 = _gather_l2(feat1, src1)
    _feat2, logits2 = _attn_layer(
        N2, D1, 256, feat1[:N2], rows2, Wq[1], Wk[1], Wv[1], Ws[1],
        bq[1], bv[1], bs[1], bn_g[1], bn_b[1], bn_m[1], bn_v[1], Wo[2], bo[2])
    return (t0, t1, logits2[0].reshape(N2, 1))


# speedup vs baseline: 74.8551x; 1.0079x over previous
import functools

import jax
import jax.numpy as jnp
from jax import lax
from jax.experimental import pallas as pl
from jax.experimental.pallas import tpu as pltpu
from jax.experimental.pallas import tpu_sc as plsc

N0, N1, N2 = 100000, 20000, 2048
N0P = 100352
D0, D1 = 16, 16
DIM, OUTC, H = 128, 64, 2
K1SEL = D0 // 2

NC, NS = 2, 16
NW = NC * NS

@functools.lru_cache(maxsize=1)
def _sc_mesh():
    return plsc.VectorSubcoreMesh(core_axis_name="c", subcore_axis_name="s")


def _wid():
    return lax.axis_index("s") * NC + lax.axis_index("c")


def _logits_body(blk, x_ref, wt_ref, b_ref, logit_ref, sig_ref):
    i = pl.program_id(0)
    z = lax.dot_general(wt_ref[...], x_ref[...],
                        (((1,), (1,)), ((), ()))) + b_ref[...]
    logit_ref[:, pl.ds(i * blk, blk)] = z
    sig_ref[:, pl.ds(i * blk, blk)] = jax.nn.sigmoid(z)


def _logits0(x, Wo0, bo0):
    blk = 2048
    grid = (N0P // blk,)
    return pl.pallas_call(
        functools.partial(_logits_body, blk),
        grid=grid,
        in_specs=[
            pl.BlockSpec((blk, DIM), lambda i: (i, 0)),
            pl.BlockSpec((1, DIM), lambda i: (0, 0)),
            pl.BlockSpec((1, 1), lambda i: (0, 0)),
        ],
        out_specs=[
            pl.BlockSpec((1, N0P), lambda i: (0, 0)),
            pl.BlockSpec((1, N0P), lambda i: (0, 0)),
        ],
        out_shape=[
            jax.ShapeDtypeStruct((1, N0P), jnp.float32),
            jax.ShapeDtypeStruct((1, N0P), jnp.float32),
        ],
    )(x, Wo0.reshape(1, DIM), bo0.reshape(1, 1))


def _topk_body(l_hbm, src_hbm, out_hbm, l_v, src_v, sel_v, tmp_v):
    wid = _wid()
    gpw = N1 // NW
    pltpu.sync_copy(l_hbm, l_v)
    pltpu.sync_copy(src_hbm.at[pl.ds(wid * gpw * D0, gpw * D0)], src_v)

    eight = jnp.full((16,), K1SEL, jnp.int32)
    seven = jnp.full((16,), K1SEL - 1, jnp.int32)
    lanes = lax.iota(jnp.int32, 16)

    def body(g, _):
        idx = src_v[pl.ds(g * D0, 16)]
        row = plsc.load_gather(l_v, [idx])
        d = wid * gpw + g
        col = plsc.load_gather(l_v, [jnp.full((16,), d, jnp.int32)])
        sim = 1.0 - jnp.abs(row - col)
        skeys, _vals = plsc.sort_key_val(sim, lanes, descending=True)
        tmp_v[...] = skeys
        t = plsc.load_gather(tmp_v, [seven])
        gt = sim > t
        n_gt = plsc.all_reduce_population_count(gt)
        eq = sim == t
        csum = lax.cumsum(jnp.where(eq, 1, 0), axis=0)
        sel = jnp.logical_or(gt, jnp.logical_and(eq, csum <= (eight - n_gt)))
        plsc.store_compressed(sel_v.at[pl.ds(g * K1SEL, 16)], idx, mask=sel)
        return _

    lax.fori_loop(0, gpw, body, 0)
    opw = gpw * K1SEL
    pltpu.sync_copy(sel_v.at[pl.ds(0, opw)], out_hbm.at[pl.ds(wid * opw, opw)])


def _topk_select(l, src0):
    gpw = N1 // NW
    f = pl.kernel(
        _topk_body,
        mesh=_sc_mesh(),
        compiler_params=pltpu.CompilerParams(needs_layout_passes=False),
        out_type=jax.ShapeDtypeStruct((N1 * K1SEL,), jnp.int32),
        scratch_types=[
            pltpu.VMEM((N0P,), jnp.float32),
            pltpu.VMEM((gpw * D0,), jnp.int32),
            pltpu.VMEM((gpw * K1SEL + 16,), jnp.int32),
            pltpu.VMEM((16,), jnp.float32),
        ],
    )
    return f(l, src0)


def _make_gather(n_rows_table, n_idx, chunk):
    bpw = n_idx // NW
    assert bpw % chunk == 0 and chunk % 8 == 0

    nch = bpw // chunk

    def body(table_hbm, idx_hbm, out_hbm, idx_v, rows_a, rows_b, sem_a, sem_b):
        wid = _wid()
        base = wid * bpw
        pltpu.sync_copy(idx_hbm.at[pl.ds(base, bpw)], idx_v)

        def gather(c, buf, sem):
            pltpu.async_copy(
                table_hbm.at[idx_v.at[pl.ds(c * chunk, chunk)]], buf, sem)

        def gwait(c, buf, sem):
            pltpu.make_async_copy(
                table_hbm.at[idx_v.at[pl.ds(c * chunk, chunk)]], buf, sem
            ).wait()

        gather(0, rows_a, sem_a)

        def step(c, carry):
            even = (c % 2) == 0

            @pl.when(jnp.logical_and(c + 1 < nch, even))
            def _pf_b():
                gather(c + 1, rows_b, sem_b)

            @pl.when(jnp.logical_and(c + 1 < nch, jnp.logical_not(even)))
            def _pf_a():
                gather(c + 1, rows_a, sem_a)

            @pl.when(even)
            def _drain_a():
                gwait(c, rows_a, sem_a)
                pltpu.sync_copy(
                    rows_a, out_hbm.at[pl.ds(base + c * chunk, chunk), :])

            @pl.when(jnp.logical_not(even))
            def _drain_b():
                gwait(c, rows_b, sem_b)
                pltpu.sync_copy(
                    rows_b, out_hbm.at[pl.ds(base + c * chunk, chunk), :])
            return carry

        lax.fori_loop(0, nch, step, 0)

    def run(table, idx):
        f = pl.kernel(
            body,
            mesh=_sc_mesh(),
            compiler_params=pltpu.CompilerParams(needs_layout_passes=False),
            out_type=jax.ShapeDtypeStruct((n_idx, DIM), jnp.float32),
            scratch_types=[
                pltpu.VMEM((bpw,), jnp.int32),
                pltpu.VMEM((chunk, DIM), jnp.float32),
                pltpu.VMEM((chunk, DIM), jnp.float32),
                pltpu.SemaphoreType.DMA,
                pltpu.SemaphoreType.DMA,
            ],
        )
        return f(table, idx)

    return run


_gather_l1 = _make_gather(N0, N1 * K1SEL, 200)
_gather_l2 = _make_gather(N1, N2 * D1, 256)


def _attn_body(deg, xd_ref, rows_ref, wq_ref, wk_ref, wv_ref, ws_ref,
               bq_ref, bv_ref, bs_ref, bng_ref, bnb_ref, bnm_ref, bnv_ref,
               wo_ref, bo_ref, feat_ref, logit_ref):
    xd = xd_ref[...]
    bd = xd.shape[0]
    q = xd @ wq_ref[...] + bq_ref[...]
    rows = rows_ref[...]
    rows3 = rows.reshape(bd, deg, DIM)
    vv = rows @ wv_ref[...]
    inv_sqrt = 1.0 / jnp.sqrt(float(OUTC))
    scores = []
    for h in range(H):
        sl = slice(h * OUTC, (h + 1) * OUTC)
        qh = q[:, sl]
        qWh = lax.dot_general(
            qh, wk_ref[...][:, sl], (((1,), (1,)), ((), ())))
        scores.append((rows3 * qWh[:, None, :]).sum(-1))
    s01 = jnp.concatenate(scores, axis=-1) * inv_sqrt
    m0 = s01[:, :deg].max(axis=-1, keepdims=True)
    m1 = s01[:, deg:].max(axis=-1, keepdims=True)
    mC = jnp.concatenate([jnp.broadcast_to(m0, (bd, deg)),
                          jnp.broadcast_to(m1, (bd, deg))], axis=-1)
    ex = jnp.exp(s01 - mC)
    d0 = ex[:, :deg].sum(axis=-1, keepdims=True)
    d1 = ex[:, deg:].sum(axis=-1, keepdims=True)
    dC = jnp.concatenate([jnp.broadcast_to(d0, (bd, deg)),
                          jnp.broadcast_to(d1, (bd, deg))], axis=-1)
    alpha = ex / (dC + 1e-16)
    outs = []
    for h in range(H):
        sl = slice(h * OUTC, (h + 1) * OUTC)
        ah = alpha[:, h * deg:(h + 1) * deg]
        vvh = vv[:, sl].reshape(bd, deg, OUTC)
        outs.append((ah[:, :, None] * vvh).sum(1))
    out = jnp.concatenate(outs, axis=-1) + bv_ref[...]
    out = out + xd @ ws_ref[...] + bs_ref[...]
    z = (out - bnm_ref[...]) / jnp.sqrt(bnv_ref[...] + 1e-5)
    z = z * bng_ref[...] + bnb_ref[...]
    sp = jnp.maximum(z, 0.0) + jnp.log1p(jnp.exp(-jnp.abs(z)))
    feat = z * jnp.tanh(sp)
    feat_ref[...] = feat
    logit_ref[:, pl.ds(pl.program_id(0) * bd, bd)] = lax.dot_general(
        wo_ref[...], feat, (((1,), (1,)), ((), ()))) + bo_ref[...]


def _attn_layer(n_dst, deg, blk, x_dst_src, rows, Wq, Wk, Wv, Ws, bq, bv,
                bs, bng, bnb, bnm, bnv, Wo, bo):
    nblk = (n_dst + blk - 1) // blk
    npad = nblk * blk
    grid = (nblk,)
    wspec = pl.BlockSpec((DIM, DIM), lambda i: (0, 0))
    bspec = pl.BlockSpec((1, DIM), lambda i: (0, 0))
    return pl.pallas_call(
        functools.partial(_attn_body, deg),
        grid=grid,
        in_specs=[
            pl.BlockSpec((blk, DIM), lambda i: (i, 0)),
            pl.BlockSpec((blk * deg, DIM), lambda i: (i, 0)),
            wspec, wspec, wspec, wspec,
            bspec, bspec, bspec, bspec, bspec, bspec, bspec,
            pl.BlockSpec((1, DIM), lambda i: (0, 0)),
            pl.BlockSpec((1, 1), lambda i: (0, 0)),
        ],
        out_specs=[
            pl.BlockSpec((blk, DIM), lambda i: (i, 0)),
            pl.BlockSpec((1, npad), lambda i: (0, 0)),
        ],
        out_shape=[
            jax.ShapeDtypeStruct((n_dst, DIM), jnp.float32),
            jax.ShapeDtypeStruct((1, npad), jnp.float32),
        ],
    )(x_dst_src, rows, Wq, Wk, Wv, Ws,
      bq.reshape(1, DIM), bv.reshape(1, DIM), bs.reshape(1, DIM),
      bng.reshape(1, DIM), bnb.reshape(1, DIM), bnm.reshape(1, DIM),
      bnv.reshape(1, DIM), Wo.reshape(1, DIM), bo.reshape(1, 1))


def kernel(x, edge_index_0, edge_index_1, Wq, bq, Wk, bk, Wv, bv, Ws, bs,
           bn_g, bn_b, bn_m, bn_v, Wo, bo):
    src0 = edge_index_0[0]
    src1 = edge_index_1[0]

    logits0, l = _logits0(x, Wo[0], bo[0])
    t0 = logits0[0, :N2].reshape(N2, 1)

    sel_src = _topk_select(l.reshape(-1), src0)
    rows1 = _gather_l1(x, sel_src)

    feat1, logits1 = _attn_layer(
        N1, K1SEL, 512, x, rows1, Wq[0], Wk[0], Wv[0], Ws[0],
        bq[0], bv[0], bs[0], bn_g[0], bn_b[0], bn_m[0], bn_v[0], Wo[1], bo[1])
    t1 = logits1[0, :N2].reshape(N2, 1)

    rows2 = _gather_l2(feat1, src1)
    _feat2, logits2 = _attn_layer(
        N2, D1, 256, feat1, rows2, Wq[1], Wk[1], Wv[1], Ws[1],
        bq[1], bv[1], bs[1], bn_g[1], bn_b[1], bn_m[1], bn_v[1], Wo[2], bo[2])
    return (t0, t1, logits2[0].reshape(N2, 1))
